# baseline (speedup 1.0000x reference)
import jax
import jax.numpy as jnp
from jax import lax
from jax.experimental import pallas as pl
from jax.experimental.pallas import tpu as pltpu

M = 2048
D = 128
N = 8192
TN = 512
NB = N // TN


def _body(q_ref, v_ref, k_ref, out_ref, cmacc, acc):
    b = pl.program_id(0)

    q = q_ref[...]
    ss = jnp.sum(q * q, axis=1, keepdims=True)
    qn = q / jnp.maximum(jnp.sqrt(ss), 1e-12)
    score = lax.dot_general(
        k_ref[...], qn, (((1,), (1,)), ((), ())),
        preferred_element_type=jnp.float32)
    smax = jnp.max(score, axis=0, keepdims=True)
    iota0 = lax.broadcasted_iota(jnp.int32, (M, TN), 0)
    g = jnp.min(jnp.where(score == smax, iota0, M),
                axis=0, keepdims=True)
    onehot = (iota0 == g).astype(jnp.bfloat16)
    e = jnp.exp(smax)
    ecol = jnp.transpose(e, (1, 0))
    uv = v_ref[...] * ecol
    uv_hi = uv.astype(jnp.bfloat16)
    uv_lo = (uv - uv_hi.astype(jnp.float32)).astype(jnp.bfloat16)
    contrib = lax.dot_general(
        onehot, uv_hi, (((1,), (0,)), ((), ())),
        preferred_element_type=jnp.float32)
    contrib += lax.dot_general(
        onehot, uv_lo, (((1,), (0,)), ((), ())),
        preferred_element_type=jnp.float32)

    @pl.when(b == 0)
    def _():
        cmacc[...] = score
        acc[...] = contrib

    @pl.when(b > 0)
    def _():
        cmacc[...] = jnp.maximum(cmacc[...], score)
        acc[...] += contrib

    @pl.when(b == NB - 1)
    def _():
        c = jnp.max(cmacc[...], axis=1, keepdims=True)
        mem = acc[...] * jnp.exp(-c) + k_ref[...]
        nn = jnp.sqrt(jnp.sum(mem * mem, axis=1, keepdims=True))
        out_ref[...] = mem / jnp.maximum(nn, 1e-12)


def kernel(keys, query, value):
    qf = jnp.transpose(query, (0, 2, 3, 1)).reshape(N, D)
    vf = jnp.transpose(value, (0, 2, 3, 1)).reshape(N, D)

    out = pl.pallas_call(
        _body,
        grid=(NB,),
        in_specs=[
            pl.BlockSpec((TN, D), lambda b: (b, 0)),
            pl.BlockSpec((TN, D), lambda b: (b, 0)),
            pl.BlockSpec((M, D), lambda b: (0, 0)),
        ],
        out_specs=pl.BlockSpec((M, D), lambda b: (0, 0)),
        out_shape=jax.ShapeDtypeStruct((M, D), jnp.float32),
        scratch_shapes=[
            pltpu.VMEM((M, TN), jnp.float32),
            pltpu.VMEM((M, D), jnp.float32),
        ],
    )(qf, vf, keys)
    return out


# speedup vs baseline: 1.2894x; 1.2894x over previous
import jax
import jax.numpy as jnp
from jax import lax
from jax.experimental import pallas as pl
from jax.experimental.pallas import tpu as pltpu

M = 2048
D = 128
N = 8192
TN = 512
NB = N // TN


def _body(q_ref, v_ref, k_ref, out_ref, cmacc, acc):
    b = pl.program_id(0)

    q = q_ref[...]
    ss = jnp.sum(q * q, axis=1, keepdims=True)
    qn = q / jnp.maximum(jnp.sqrt(ss), 1e-12)
    score = lax.dot_general(
        k_ref[...], qn, (((1,), (1,)), ((), ())),
        preferred_element_type=jnp.float32)
    smax = jnp.max(score, axis=0, keepdims=True)
    iota0 = lax.broadcasted_iota(jnp.int32, (M, TN), 0)
    g = jnp.min(jnp.where(score == smax, iota0, M),
                axis=0, keepdims=True)
    e = jnp.exp(smax)
    woh = jnp.where(iota0 == g, e, 0.0)
    contrib = lax.dot_general(
        woh, v_ref[...], (((1,), (0,)), ((), ())),
        preferred_element_type=jnp.float32)

    @pl.when(b == 0)
    def _():
        cmacc[...] = score
        acc[...] = contrib

    @pl.when(b > 0)
    def _():
        cmacc[...] = jnp.maximum(cmacc[...], score)
        acc[...] += contrib

    @pl.when(b == NB - 1)
    def _():
        c = jnp.max(cmacc[...], axis=1, keepdims=True)
        mem = acc[...] * jnp.exp(-c) + k_ref[...]
        nn = jnp.sqrt(jnp.sum(mem * mem, axis=1, keepdims=True))
        out_ref[...] = mem / jnp.maximum(nn, 1e-12)


def kernel(keys, query, value):
    qf = jnp.transpose(query, (0, 2, 3, 1)).reshape(N, D)
    vf = jnp.transpose(value, (0, 2, 3, 1)).reshape(N, D)

    out = pl.pallas_call(
        _body,
        grid=(NB,),
        in_specs=[
            pl.BlockSpec((TN, D), lambda b: (b, 0)),
            pl.BlockSpec((TN, D), lambda b: (b, 0)),
            pl.BlockSpec((M, D), lambda b: (0, 0)),
        ],
        out_specs=pl.BlockSpec((M, D), lambda b: (0, 0)),
        out_shape=jax.ShapeDtypeStruct((M, D), jnp.float32),
        scratch_shapes=[
            pltpu.VMEM((M, TN), jnp.float32),
            pltpu.VMEM((M, D), jnp.float32),
        ],
    )(qf, vf, keys)
    return out


# speedup vs baseline: 1.5912x; 1.2341x over previous
import jax
import jax.numpy as jnp
from jax import lax
from jax.experimental import pallas as pl
from jax.experimental.pallas import tpu as pltpu

M = 2048
D = 128
N = 8192
TN = 1024
NB = N // TN


def _body(q_ref, v_ref, k_ref, out_ref, cmacc, acc):
    b = pl.program_id(0)

    q = q_ref[...]
    ss = jnp.sum(q * q, axis=1, keepdims=True)
    qn = q / jnp.maximum(jnp.sqrt(ss), 1e-12)
    score = lax.dot_general(
        k_ref[...], qn, (((1,), (1,)), ((), ())),
        preferred_element_type=jnp.float32)
    smax = jnp.max(score, axis=0, keepdims=True)
    iota0 = lax.broadcasted_iota(jnp.int32, (M, TN), 0)
    g = jnp.min(jnp.where(score == smax, iota0, M),
                axis=0, keepdims=True)
    e = jnp.exp(smax)
    woh = jnp.where(iota0 == g, e, 0.0)
    contrib = lax.dot_general(
        woh, v_ref[...], (((1,), (0,)), ((), ())),
        preferred_element_type=jnp.float32)

    @pl.when(b == 0)
    def _():
        cmacc[...] = score
        acc[...] = contrib

    @pl.when(b > 0)
    def _():
        cmacc[...] = jnp.maximum(cmacc[...], score)
        acc[...] += contrib

    @pl.when(b == NB - 1)
    def _():
        c = jnp.max(cmacc[...], axis=1, keepdims=True)
        mem = acc[...] * jnp.exp(-c) + k_ref[...]
        nn = jnp.sqrt(jnp.sum(mem * mem, axis=1, keepdims=True))
        out_ref[...] = mem / jnp.maximum(nn, 1e-12)


def kernel(keys, query, value):
    qf = jnp.transpose(query, (0, 2, 3, 1)).reshape(N, D)
    vf = jnp.transpose(value, (0, 2, 3, 1)).reshape(N, D)

    out = pl.pallas_call(
        _body,
        grid=(NB,),
        in_specs=[
            pl.BlockSpec((TN, D), lambda b: (b, 0)),
            pl.BlockSpec((TN, D), lambda b: (b, 0)),
            pl.BlockSpec((M, D), lambda b: (0, 0)),
        ],
        out_specs=pl.BlockSpec((M, D), lambda b: (0, 0)),
        out_shape=jax.ShapeDtypeStruct((M, D), jnp.float32),
        scratch_shapes=[
            pltpu.VMEM((M, TN), jnp.float32),
            pltpu.VMEM((M, D), jnp.float32),
        ],
    )(qf, vf, keys)
    return out


# speedup vs baseline: 1.7635x; 1.1083x over previous
import jax
import jax.numpy as jnp
from jax import lax
from jax.experimental import pallas as pl
from jax.experimental.pallas import tpu as pltpu

M = 2048
D = 128
N = 8192
TN = 2048
NB = N // TN


def _body(q_ref, v_ref, k_ref, out_ref, cmacc, acc):
    b = pl.program_id(0)

    q = q_ref[...]
    ss = jnp.sum(q * q, axis=1, keepdims=True)
    qn = q / jnp.maximum(jnp.sqrt(ss), 1e-12)
    score = lax.dot_general(
        k_ref[...], qn, (((1,), (1,)), ((), ())),
        preferred_element_type=jnp.float32)
    smax = jnp.max(score, axis=0, keepdims=True)
    iota0 = lax.broadcasted_iota(jnp.int32, (M, TN), 0)
    g = jnp.min(jnp.where(score == smax, iota0, M),
                axis=0, keepdims=True)
    e = jnp.exp(smax)
    woh = jnp.where(iota0 == g, e, 0.0)
    contrib = lax.dot_general(
        woh, v_ref[...], (((1,), (0,)), ((), ())),
        preferred_element_type=jnp.float32)

    @pl.when(b == 0)
    def _():
        cmacc[...] = score
        acc[...] = contrib

    @pl.when(b > 0)
    def _():
        cmacc[...] = jnp.maximum(cmacc[...], score)
        acc[...] += contrib

    @pl.when(b == NB - 1)
    def _():
        c = jnp.max(cmacc[...], axis=1, keepdims=True)
        mem = acc[...] * jnp.exp(-c) + k_ref[...]
        nn = jnp.sqrt(jnp.sum(mem * mem, axis=1, keepdims=True))
        out_ref[...] = mem / jnp.maximum(nn, 1e-12)


def kernel(keys, query, value):
    qf = jnp.transpose(query, (0, 2, 3, 1)).reshape(N, D)
    vf = jnp.transpose(value, (0, 2, 3, 1)).reshape(N, D)

    out = pl.pallas_call(
        _body,
        grid=(NB,),
        in_specs=[
            pl.BlockSpec((TN, D), lambda b: (b, 0)),
            pl.BlockSpec((TN, D), lambda b: (b, 0)),
            pl.BlockSpec((M, D), lambda b: (0, 0)),
        ],
        out_specs=pl.BlockSpec((M, D), lambda b: (0, 0)),
        out_shape=jax.ShapeDtypeStruct((M, D), jnp.float32),
        scratch_shapes=[
            pltpu.VMEM((M, TN), jnp.float32),
            pltpu.VMEM((M, D), jnp.float32),
        ],
    )(qf, vf, keys)
    return out


# speedup vs baseline: 1.9144x; 1.0856x over previous
import jax
import jax.numpy as jnp
from jax import lax
from jax.experimental import pallas as pl
from jax.experimental.pallas import tpu as pltpu

M = 2048
D = 128
N = 8192
TN = 2048
NB = N // TN


def _body(q_ref, v_ref, k_ref, out_ref, cmacc, acc):
    b = pl.program_id(0)

    q = q_ref[...]
    ss = jnp.sum(q * q, axis=1, keepdims=True)
    qn = q / jnp.maximum(jnp.sqrt(ss), 1e-12)
    score = lax.dot_general(
        k_ref[...], qn, (((1,), (1,)), ((), ())),
        preferred_element_type=jnp.float32)
    smax = jnp.max(score, axis=0, keepdims=True)
    iota0 = lax.broadcasted_iota(jnp.int32, (M, TN), 0)
    g = jnp.min(jnp.where(score == smax, iota0, M),
                axis=0, keepdims=True)
    e = jnp.exp(smax)
    woh = jnp.where(iota0 == g, e, 0.0)
    contrib = lax.dot_general(
        woh, v_ref[...], (((1,), (0,)), ((), ())),
        preferred_element_type=jnp.float32)

    cpart = jnp.max(score, axis=1, keepdims=True)

    @pl.when(b == 0)
    def _():
        cmacc[...] = cpart
        acc[...] = contrib

    @pl.when(b > 0)
    def _():
        cmacc[...] = jnp.maximum(cmacc[...], cpart)
        acc[...] += contrib

    @pl.when(b == NB - 1)
    def _():
        c = jnp.maximum(cmacc[...], cpart)
        mem = acc[...] * jnp.exp(-c) + k_ref[...]
        nn = jnp.sqrt(jnp.sum(mem * mem, axis=1, keepdims=True))
        out_ref[...] = mem / jnp.maximum(nn, 1e-12)


def kernel(keys, query, value):
    qf = jnp.transpose(query, (0, 2, 3, 1)).reshape(N, D)
    vf = jnp.transpose(value, (0, 2, 3, 1)).reshape(N, D)

    out = pl.pallas_call(
        _body,
        grid=(NB,),
        in_specs=[
            pl.BlockSpec((TN, D), lambda b: (b, 0)),
            pl.BlockSpec((TN, D), lambda b: (b, 0)),
            pl.BlockSpec((M, D), lambda b: (0, 0)),
        ],
        out_specs=pl.BlockSpec((M, D), lambda b: (0, 0)),
        out_shape=jax.ShapeDtypeStruct((M, D), jnp.float32),
        scratch_shapes=[
            pltpu.VMEM((M, 1), jnp.float32),
            pltpu.VMEM((M, D), jnp.float32),
        ],
    )(qf, vf, keys)
    return out


# speedup vs baseline: 2.0117x; 1.0508x over previous
import jax
import jax.numpy as jnp
from jax import lax
from jax.experimental import pallas as pl
from jax.experimental.pallas import tpu as pltpu

M = 2048
D = 128
N = 8192
TN = 4096
NB = N // TN


def _body(q_ref, v_ref, k_ref, out_ref, cmacc, acc):
    b = pl.program_id(0)

    q = q_ref[...]
    ss = jnp.sum(q * q, axis=1, keepdims=True)
    qn = q / jnp.maximum(jnp.sqrt(ss), 1e-12)
    score = lax.dot_general(
        k_ref[...], qn, (((1,), (1,)), ((), ())),
        preferred_element_type=jnp.float32)
    smax = jnp.max(score, axis=0, keepdims=True)
    iota0 = lax.broadcasted_iota(jnp.int32, (M, TN), 0)
    g = jnp.min(jnp.where(score == smax, iota0, M),
                axis=0, keepdims=True)
    e = jnp.exp(smax)
    woh = jnp.where(iota0 == g, e, 0.0)
    contrib = lax.dot_general(
        woh, v_ref[...], (((1,), (0,)), ((), ())),
        preferred_element_type=jnp.float32)

    cpart = jnp.max(score, axis=1, keepdims=True)

    @pl.when(b == 0)
    def _():
        cmacc[...] = cpart
        acc[...] = contrib

    @pl.when(b > 0)
    def _():
        cmacc[...] = jnp.maximum(cmacc[...], cpart)
        acc[...] += contrib

    @pl.when(b == NB - 1)
    def _():
        c = jnp.maximum(cmacc[...], cpart)
        mem = acc[...] * jnp.exp(-c) + k_ref[...]
        nn = jnp.sqrt(jnp.sum(mem * mem, axis=1, keepdims=True))
        out_ref[...] = mem / jnp.maximum(nn, 1e-12)


def kernel(keys, query, value):
    qf = jnp.transpose(query, (0, 2, 3, 1)).reshape(N, D)
    vf = jnp.transpose(value, (0, 2, 3, 1)).reshape(N, D)

    out = pl.pallas_call(
        _body,
        grid=(NB,),
        in_specs=[
            pl.BlockSpec((TN, D), lambda b: (b, 0)),
            pl.BlockSpec((TN, D), lambda b: (b, 0)),
            pl.BlockSpec((M, D), lambda b: (0, 0)),
        ],
        out_specs=pl.BlockSpec((M, D), lambda b: (0, 0)),
        out_shape=jax.ShapeDtypeStruct((M, D), jnp.float32),
        scratch_shapes=[
            pltpu.VMEM((M, 1), jnp.float32),
            pltpu.VMEM((M, D), jnp.float32),
        ],
    )(qf, vf, keys)
    return out
